# Initial kernel scaffold; baseline (speedup 1.0000x reference)
import functools

import jax
import jax.numpy as jnp
from jax import lax
from jax.experimental import pallas as pl
from jax.experimental.pallas import tpu as pltpu
from jax.experimental.pallas import tpu_sc as plsc

_N = 10000
_E = 320000
_D = 128
_B = 8

_NC = 2
_NS = 16
_NT = _NC * _NS
_NP = 10240
_RPT = _NP // _NS
_EPT = _E // _NT
_CH = 80
_STEPS = _EPT // _CH


def _make_sc_agg(with_deg):
  out_type = [jax.ShapeDtypeStruct((_NC, _NP, _D), jnp.float32)]
  scratch = [
      pltpu.VMEM((_CH,), jnp.int32),
      pltpu.VMEM((_CH,), jnp.int32),
      pltpu.VMEM((_CH, _D), jnp.float32),
      pltpu.VMEM_SHARED((_NP, _D), jnp.float32),
      pltpu.SemaphoreType.DMA,
  ]
  if with_deg:
    out_type.append(jax.ShapeDtypeStruct((_NC, _NP, 16), jnp.float32))
    scratch.append(pltpu.VMEM((_CH, 16), jnp.float32))
    scratch.append(pltpu.VMEM_SHARED((_NP, 16), jnp.float32))

  def body(*refs):
    if with_deg:
      (h_hbm, src_hbm, dst_hbm, zf_hbm, z16_hbm, ones_hbm,
       out_acc, out_deg, src_v, dst_v, rows_v, acc_s, sem,
       ones_v, deg_s) = refs
    else:
      (h_hbm, src_hbm, dst_hbm, zf_hbm,
       out_acc, src_v, dst_v, rows_v, acc_s, sem) = refs
    c = lax.axis_index("c")
    s = lax.axis_index("s")
    wid = c * _NS + s
    pltpu.sync_copy(zf_hbm, acc_s.at[pl.ds(s * _RPT, _RPT)])
    if with_deg:
      pltpu.sync_copy(z16_hbm, deg_s.at[pl.ds(s * _RPT, _RPT)])
      pltpu.sync_copy(ones_hbm, ones_v)
    plsc.subcore_barrier()
    base = wid * _EPT

    def step(i, carry):
      off = base + i * _CH
      pltpu.sync_copy(src_hbm.at[pl.ds(off, _CH)], src_v)
      pltpu.sync_copy(dst_hbm.at[pl.ds(off, _CH)], dst_v)
      pltpu.async_copy(h_hbm.at[src_v], rows_v, sem).wait()
      pltpu.sync_copy(rows_v, acc_s.at[dst_v], add=True)
      if with_deg:
        pltpu.sync_copy(ones_v, deg_s.at[dst_v], add=True)
      return carry

    lax.fori_loop(0, _STEPS, step, 0)
    plsc.subcore_barrier()
    pltpu.sync_copy(acc_s.at[pl.ds(s * _RPT, _RPT)],
                    out_acc.at[c, pl.ds(s * _RPT, _RPT)])
    if with_deg:
      pltpu.sync_copy(deg_s.at[pl.ds(s * _RPT, _RPT)],
                      out_deg.at[c, pl.ds(s * _RPT, _RPT)])

  mesh = plsc.VectorSubcoreMesh(core_axis_name="c", subcore_axis_name="s")
  return pl.kernel(body, out_type=tuple(out_type), mesh=mesh,
                   scratch_types=tuple(scratch))


def _dot_t(a, w):
  return lax.dot_general(a, w, (((1,), (1,)), ((), ())),
                         preferred_element_type=jnp.float32)


def _tc_layer1(accp_ref, degp_ref, x_ref, wl_ref, bl_ref, wr_ref, out_ref):
  deg = jnp.sum(degp_ref[...], axis=(0, 2))[:_N]
  inv = 1.0 / jnp.maximum(deg, 1.0)
  agg = (accp_ref[0, :_N, :] + accp_ref[1, :_N, :]) * inv[:, None]
  h = _dot_t(agg, wl_ref[...]) + bl_ref[...] + _dot_t(x_ref[...], wr_ref[...])
  out_ref[...] = jnp.maximum(h, 0.0)


def _tc_final(accp_ref, degp_ref, h1_ref, wl_ref, bl_ref, wr_ref,
              tf_ref, wt1_ref, bt1_ref, wt2_ref, bt2_ref,
              wc1_ref, bc1_ref, wc2_ref, bc2_ref, out_ref):
  deg = jnp.sum(degp_ref[...], axis=(0, 2))[:_N]
  inv = 1.0 / jnp.maximum(deg, 1.0)
  agg = (accp_ref[0, :_N, :] + accp_ref[1, :_N, :]) * inv[:, None]
  h2 = jnp.maximum(
      _dot_t(agg, wl_ref[...]) + bl_ref[...] + _dot_t(h1_ref[...], wr_ref[...]),
      0.0)
  t = _dot_t(jnp.maximum(_dot_t(tf_ref[...], wt1_ref[...]) + bt1_ref[...], 0.0),
             wt2_ref[...]) + bt2_ref[...]
  wc1 = wc1_ref[...]
  g = _dot_t(h2, wc1[:, :_D])
  cb = _dot_t(t, wc1[:, _D:]) + bc1_ref[...]
  wc2 = wc2_ref[...]
  cols = []
  for b in range(_B):
    hid = jnp.maximum(g + cb[b:b + 1, :], 0.0)
    cols.append(_dot_t(hid, wc2))
  out_ref[...] = jnp.concatenate(cols, axis=1) + bc2_ref[0, 0]


def kernel(x, edge_index, task_feat, W_l1, b_l1, W_r1, W_l2, b_l2, W_r2,
           Wt1, bt1, Wt2, bt2, Wc1, bc1, Wc2, bc2):
  src = edge_index[0]
  dst = edge_index[1]
  zf = jnp.zeros((_RPT, _D), jnp.float32)
  z16 = jnp.zeros((_RPT, 16), jnp.float32)
  ones = jnp.ones((_CH, 16), jnp.float32)

  acc1, degp = _make_sc_agg(True)(x, src, dst, zf, z16, ones)

  h1 = pl.pallas_call(
      _tc_layer1,
      out_shape=jax.ShapeDtypeStruct((_N, _D), jnp.float32),
  )(acc1, degp, x, W_l1, b_l1.reshape(1, _D), W_r1)

  acc2 = _make_sc_agg(False)(h1, src, dst, zf)

  scores_t = pl.pallas_call(
      _tc_final,
      out_shape=jax.ShapeDtypeStruct((_N, _B), jnp.float32),
  )(acc2, degp, h1, W_l2, b_l2.reshape(1, _D), W_r2,
    task_feat, Wt1, bt1.reshape(1, _D), Wt2, bt2.reshape(1, _D),
    Wc1, bc1.reshape(1, _D), Wc2, bc2.reshape(1, 1))

  return scores_t.T


# speedup vs baseline: 3.1308x; 3.1308x over previous
import functools

import jax
import jax.numpy as jnp
from jax import lax
from jax.experimental import pallas as pl
from jax.experimental.pallas import tpu as pltpu
from jax.experimental.pallas import tpu_sc as plsc

_N = 10000
_E = 320000
_D = 128
_B = 8

_NC = 2
_NS = 16
_HD = _D // _NC
_NP = 10240
_RPT = _NP // _NS
_EPT = _E // _NS
_CH = 80
_STEPS = _EPT // _CH


def _make_sc_agg(with_deg):
  out_type = [jax.ShapeDtypeStruct((_NC * _NP, _HD), jnp.float32)]
  scratch = [
      pltpu.VMEM((_CH,), jnp.int32),
      pltpu.VMEM((_CH,), jnp.int32),
      pltpu.VMEM((_CH, _HD), jnp.float32),
      pltpu.VMEM((_RPT, _HD), jnp.float32),
      pltpu.VMEM_SHARED((_NP, _HD), jnp.float32),
      pltpu.SemaphoreType.DMA,
  ]
  if with_deg:
    out_type.append(jax.ShapeDtypeStruct((_NC * _NP, 16), jnp.float32))
    scratch.append(pltpu.VMEM((_CH, 16), jnp.float32))
    scratch.append(pltpu.VMEM((_RPT, 16), jnp.float32))
    scratch.append(pltpu.VMEM_SHARED((_NP, 16), jnp.float32))

  def body(*refs):
    if with_deg:
      (hlo_hbm, hhi_hbm, src_hbm, dst_hbm, zf_hbm, z16_hbm, ones_hbm,
       out_acc, out_deg, src_v, dst_v, rows_v, stage_v, acc_s, sem,
       ones_v, dstage_v, deg_s) = refs
    else:
      (hlo_hbm, hhi_hbm, src_hbm, dst_hbm, zf_hbm,
       out_acc, src_v, dst_v, rows_v, stage_v, acc_s, sem) = refs
    c = lax.axis_index("c")
    s = lax.axis_index("s")
    pltpu.sync_copy(zf_hbm, stage_v)
    pltpu.sync_copy(stage_v, acc_s.at[pl.ds(s * _RPT, _RPT)])
    if with_deg:
      pltpu.sync_copy(ones_hbm, ones_v)
      pltpu.sync_copy(z16_hbm, dstage_v)
      pltpu.sync_copy(dstage_v, deg_s.at[pl.ds(s * _RPT, _RPT)])
    plsc.subcore_barrier()
    base = s * _EPT

    def step(i, carry):
      off = base + i * _CH
      pltpu.sync_copy(src_hbm.at[pl.ds(off, _CH)], src_v)
      pltpu.sync_copy(dst_hbm.at[pl.ds(off, _CH)], dst_v)
      @pl.when(c == 0)
      def _():
        pltpu.async_copy(hlo_hbm.at[src_v], rows_v, sem).wait()

      @pl.when(c == 1)
      def _():
        pltpu.async_copy(hhi_hbm.at[src_v], rows_v, sem).wait()

      pltpu.sync_copy(rows_v, acc_s.at[dst_v], add=True)
      if with_deg:
        pltpu.sync_copy(ones_v, deg_s.at[dst_v], add=True)
      return carry

    lax.fori_loop(0, _STEPS, step, 0)
    plsc.subcore_barrier()
    orow = c * _NP + s * _RPT
    pltpu.sync_copy(acc_s.at[pl.ds(s * _RPT, _RPT)], stage_v)
    pltpu.sync_copy(stage_v, out_acc.at[pl.ds(orow, _RPT)])
    if with_deg:
      pltpu.sync_copy(deg_s.at[pl.ds(s * _RPT, _RPT)], dstage_v)
      pltpu.sync_copy(dstage_v, out_deg.at[pl.ds(orow, _RPT)])

  mesh = plsc.VectorSubcoreMesh(core_axis_name="c", subcore_axis_name="s")
  return pl.kernel(body, out_type=tuple(out_type), mesh=mesh,
                   scratch_types=tuple(scratch),
                   compiler_params=pltpu.CompilerParams(
                       use_tc_tiling_on_sc=False))


def _dot_t(a, w):
  return lax.dot_general(a, w, (((1,), (1,)), ((), ())),
                         preferred_element_type=jnp.float32)


def _agg_mean(accp_ref, degp_ref):
  degp = degp_ref[...]
  deg = jnp.sum(degp[:_N], axis=1) * 0.0625
  inv = 1.0 / jnp.maximum(deg, 1.0)
  accp = accp_ref[...]
  agg = jnp.concatenate([accp[:_N], accp[_NP:_NP + _N]], axis=1)
  return agg * inv[:, None]


def _tc_layer1(accp_ref, degp_ref, x_ref, wl_ref, bl_ref, wr_ref, out_ref):
  agg = _agg_mean(accp_ref, degp_ref)
  h = _dot_t(agg, wl_ref[...]) + bl_ref[...] + _dot_t(x_ref[...], wr_ref[...])
  out_ref[...] = jnp.maximum(h, 0.0)


def _tc_final(accp_ref, degp_ref, h1_ref, wl_ref, bl_ref, wr_ref,
              tf_ref, wt1_ref, bt1_ref, wt2_ref, bt2_ref,
              wc1_ref, bc1_ref, wc2_ref, bc2_ref, out_ref):
  agg = _agg_mean(accp_ref, degp_ref)
  h2 = jnp.maximum(
      _dot_t(agg, wl_ref[...]) + bl_ref[...] + _dot_t(h1_ref[...], wr_ref[...]),
      0.0)
  t = _dot_t(jnp.maximum(_dot_t(tf_ref[...], wt1_ref[...]) + bt1_ref[...], 0.0),
             wt2_ref[...]) + bt2_ref[...]
  wc1 = wc1_ref[...]
  g = _dot_t(h2, wc1[:, :_D])
  cb = _dot_t(t, wc1[:, _D:]) + bc1_ref[...]
  wc2 = wc2_ref[...]
  cols = []
  for b in range(_B):
    hid = jnp.maximum(g + cb[b:b + 1, :], 0.0)
    cols.append(_dot_t(hid, wc2))
  out_ref[...] = jnp.concatenate(cols, axis=1) + bc2_ref[0, 0]


def kernel(x, edge_index, task_feat, W_l1, b_l1, W_r1, W_l2, b_l2, W_r2,
           Wt1, bt1, Wt2, bt2, Wc1, bc1, Wc2, bc2):
  src = edge_index[0]
  dst = edge_index[1]
  zf = jnp.zeros((_RPT, _HD), jnp.float32)
  z16 = jnp.zeros((_RPT, 16), jnp.float32)
  ones = jnp.ones((_CH, 16), jnp.float32)

  acc1, degp = _make_sc_agg(True)(
      x[:, :_HD], x[:, _HD:], src, dst, zf, z16, ones)

  h1 = pl.pallas_call(
      _tc_layer1,
      out_shape=jax.ShapeDtypeStruct((_N, _D), jnp.float32),
  )(acc1, degp, x, W_l1, b_l1.reshape(1, _D), W_r1)

  (acc2,) = _make_sc_agg(False)(h1[:, :_HD], h1[:, _HD:], src, dst, zf)

  scores_t = pl.pallas_call(
      _tc_final,
      out_shape=jax.ShapeDtypeStruct((_N, _B), jnp.float32),
  )(acc2, degp, h1, W_l2, b_l2.reshape(1, _D), W_r2,
    task_feat, Wt1, bt1.reshape(1, _D), Wt2, bt2.reshape(1, _D),
    Wc1, bc1.reshape(1, _D), Wc2, bc2.reshape(1, 1))

  return scores_t.T


# speedup vs baseline: 7.6545x; 2.4449x over previous
import functools

import jax
import jax.numpy as jnp
from jax import lax
from jax.experimental import pallas as pl
from jax.experimental.pallas import tpu as pltpu
from jax.experimental.pallas import tpu_sc as plsc

_N = 10000
_E = 320000
_D = 128
_B = 8

_NC = 2
_NS = 16
_HD = _D // _NC
_NP = 10240
_RPT = _NP // _NS
_EPT = _E // _NS
_CH = 80
_STEPS = _EPT // _CH
_NB = 5
_BS = _STEPS // _NB


def _make_sc_agg(with_deg):
  out_type = [jax.ShapeDtypeStruct((_NC * _NP, _HD), jnp.float32)]
  scratch = [
      pltpu.VMEM((_BS, _CH), jnp.int32),
      pltpu.VMEM((_BS, _CH), jnp.int32),
      pltpu.VMEM((_CH, _HD), jnp.float32),
      pltpu.VMEM((_CH, _HD), jnp.float32),
      pltpu.VMEM((_RPT, _HD), jnp.float32),
      pltpu.VMEM_SHARED((_NP, _HD), jnp.float32),
      pltpu.SemaphoreType.DMA,
      pltpu.SemaphoreType.DMA,
  ]
  if with_deg:
    out_type.append(jax.ShapeDtypeStruct((_NC * _NP, 16), jnp.float32))
    scratch.append(pltpu.VMEM((_CH, 16), jnp.float32))
    scratch.append(pltpu.VMEM((_RPT, 16), jnp.float32))
    scratch.append(pltpu.VMEM_SHARED((_NP, 16), jnp.float32))

  def body(*refs):
    if with_deg:
      (hlo_hbm, hhi_hbm, src_hbm, dst_hbm, zf_hbm, z16_hbm, ones_hbm,
       out_acc, out_deg, src_v, dst_v, rows0_v, rows1_v, stage_v, acc_s,
       sem0, sem1, ones_v, dstage_v, deg_s) = refs
    else:
      (hlo_hbm, hhi_hbm, src_hbm, dst_hbm, zf_hbm,
       out_acc, src_v, dst_v, rows0_v, rows1_v, stage_v, acc_s,
       sem0, sem1) = refs
    c = lax.axis_index("c")
    s = lax.axis_index("s")
    pltpu.sync_copy(zf_hbm, stage_v)
    pltpu.sync_copy(stage_v, acc_s.at[pl.ds(s * _RPT, _RPT)])
    if with_deg:
      pltpu.sync_copy(ones_hbm, ones_v)
      pltpu.sync_copy(z16_hbm, dstage_v)
      pltpu.sync_copy(dstage_v, deg_s.at[pl.ds(s * _RPT, _RPT)])
    plsc.subcore_barrier()

    def pipeline(h_hbm):
      for b in range(_NB):
        blk = s * _NB + b
        pltpu.sync_copy(src_hbm.at[blk], src_v)
        pltpu.sync_copy(dst_hbm.at[blk], dst_v)
        pltpu.async_copy(h_hbm.at[src_v.at[0]], rows0_v, sem0)
        pltpu.async_copy(h_hbm.at[src_v.at[1]], rows1_v, sem1)

        def grp(i, carry):
          g = 2 * i
          pltpu.make_async_copy(h_hbm.at[src_v.at[0]], rows0_v, sem0).wait()
          pltpu.sync_copy(rows0_v, acc_s.at[dst_v.at[g]], add=True)
          if with_deg:
            pltpu.sync_copy(ones_v, deg_s.at[dst_v.at[g]], add=True)

          @pl.when(g + 2 < _BS)
          def _():
            pltpu.async_copy(h_hbm.at[src_v.at[g + 2]], rows0_v, sem0)

          pltpu.make_async_copy(h_hbm.at[src_v.at[1]], rows1_v, sem1).wait()
          pltpu.sync_copy(rows1_v, acc_s.at[dst_v.at[g + 1]], add=True)
          if with_deg:
            pltpu.sync_copy(ones_v, deg_s.at[dst_v.at[g + 1]], add=True)

          @pl.when(g + 3 < _BS)
          def _():
            pltpu.async_copy(h_hbm.at[src_v.at[g + 3]], rows1_v, sem1)
          return carry

        lax.fori_loop(0, _BS // 2, grp, 0)

    @pl.when(c == 0)
    def _():
      pipeline(hlo_hbm)

    @pl.when(c == 1)
    def _():
      pipeline(hhi_hbm)
    plsc.subcore_barrier()
    orow = c * _NP + s * _RPT
    pltpu.sync_copy(acc_s.at[pl.ds(s * _RPT, _RPT)], stage_v)
    pltpu.sync_copy(stage_v, out_acc.at[pl.ds(orow, _RPT)])
    if with_deg:
      pltpu.sync_copy(deg_s.at[pl.ds(s * _RPT, _RPT)], dstage_v)
      pltpu.sync_copy(dstage_v, out_deg.at[pl.ds(orow, _RPT)])

  mesh = plsc.VectorSubcoreMesh(core_axis_name="c", subcore_axis_name="s")
  return pl.kernel(body, out_type=tuple(out_type), mesh=mesh,
                   scratch_types=tuple(scratch),
                   compiler_params=pltpu.CompilerParams(
                       use_tc_tiling_on_sc=False))


def _dot_t(a, w):
  return lax.dot_general(a, w, (((1,), (1,)), ((), ())),
                         preferred_element_type=jnp.float32)


def _agg_mean(accp_ref, degp_ref):
  degp = degp_ref[...]
  deg = jnp.sum(degp[:_N], axis=1) * 0.0625
  inv = 1.0 / jnp.maximum(deg, 1.0)
  accp = accp_ref[...]
  agg = jnp.concatenate([accp[:_N], accp[_NP:_NP + _N]], axis=1)
  return agg * inv[:, None]


def _tc_layer1(accp_ref, degp_ref, x_ref, wl_ref, bl_ref, wr_ref, out_ref):
  agg = _agg_mean(accp_ref, degp_ref)
  h = _dot_t(agg, wl_ref[...]) + bl_ref[...] + _dot_t(x_ref[...], wr_ref[...])
  out_ref[...] = jnp.maximum(h, 0.0)


def _tc_final(accp_ref, degp_ref, h1_ref, wl_ref, bl_ref, wr_ref,
              tf_ref, wt1_ref, bt1_ref, wt2_ref, bt2_ref,
              wc1_ref, bc1_ref, wc2_ref, bc2_ref, out_ref):
  agg = _agg_mean(accp_ref, degp_ref)
  h2 = jnp.maximum(
      _dot_t(agg, wl_ref[...]) + bl_ref[...] + _dot_t(h1_ref[...], wr_ref[...]),
      0.0)
  t = _dot_t(jnp.maximum(_dot_t(tf_ref[...], wt1_ref[...]) + bt1_ref[...], 0.0),
             wt2_ref[...]) + bt2_ref[...]
  wc1 = wc1_ref[...]
  g = _dot_t(h2, wc1[:, :_D])
  cb = _dot_t(t, wc1[:, _D:]) + bc1_ref[...]
  wc2 = wc2_ref[...]
  cols = []
  for b in range(_B):
    hid = jnp.maximum(g + cb[b:b + 1, :], 0.0)
    cols.append(_dot_t(hid, wc2))
  out_ref[...] = jnp.concatenate(cols, axis=1) + bc2_ref[0, 0]


def kernel(x, edge_index, task_feat, W_l1, b_l1, W_r1, W_l2, b_l2, W_r2,
           Wt1, bt1, Wt2, bt2, Wc1, bc1, Wc2, bc2):
  src = edge_index[0].reshape(_NS * _NB, _BS, _CH)
  dst = edge_index[1].reshape(_NS * _NB, _BS, _CH)
  zf = jnp.zeros((_RPT, _HD), jnp.float32)
  z16 = jnp.zeros((_RPT, 16), jnp.float32)
  ones = jnp.ones((_CH, 16), jnp.float32)

  acc1, degp = _make_sc_agg(True)(
      x[:, :_HD], x[:, _HD:], src, dst, zf, z16, ones)

  h1 = pl.pallas_call(
      _tc_layer1,
      out_shape=jax.ShapeDtypeStruct((_N, _D), jnp.float32),
  )(acc1, degp, x, W_l1, b_l1.reshape(1, _D), W_r1)

  (acc2,) = _make_sc_agg(False)(h1[:, :_HD], h1[:, _HD:], src, dst, zf)

  scores_t = pl.pallas_call(
      _tc_final,
      out_shape=jax.ShapeDtypeStruct((_N, _B), jnp.float32),
  )(acc2, degp, h1, W_l2, b_l2.reshape(1, _D), W_r2,
    task_feat, Wt1, bt1.reshape(1, _D), Wt2, bt2.reshape(1, _D),
    Wc1, bc1.reshape(1, _D), Wc2, bc2.reshape(1, 1))

  return scores_t.T


# speedup vs baseline: 10.1328x; 1.3238x over previous
import functools

import jax
import jax.numpy as jnp
from jax import lax
from jax.experimental import pallas as pl
from jax.experimental.pallas import tpu as pltpu
from jax.experimental.pallas import tpu_sc as plsc

_N = 10000
_E = 320000
_D = 128
_B = 8

_NC = 2
_NS = 16
_HD = _D // _NC
_NP = 10240
_RPT = _NP // _NS
_EPT = _E // _NS
_CH = 80
_STEPS = _EPT // _CH
_NB = 5
_BS = _STEPS // _NB
_NRB = 5
_SRG = 128


def _make_sc_agg(with_deg):
  out_type = [jax.ShapeDtypeStruct((_NC * _NP, _HD), jnp.float32)]
  scratch = (
      [pltpu.VMEM((_BS, _CH), jnp.int32),
       pltpu.VMEM((_BS, _CH), jnp.int32)]
      + [pltpu.VMEM((_CH, _HD), jnp.float32)] * _NRB
      + [pltpu.VMEM((_SRG, _HD), jnp.float32),
         pltpu.VMEM_SHARED((_NP, _HD), jnp.float32)]
      + [pltpu.SemaphoreType.DMA] * _NRB
  )
  if with_deg:
    out_type.append(jax.ShapeDtypeStruct((_NC * _NP, 16), jnp.float32))
    scratch.append(pltpu.VMEM((_CH, 16), jnp.float32))
    scratch.append(pltpu.VMEM((_SRG, 16), jnp.float32))
    scratch.append(pltpu.VMEM_SHARED((_NP, 16), jnp.float32))

  def body(*refs):
    if with_deg:
      (hlo_hbm, hhi_hbm, src_hbm, dst_hbm, zf_hbm, z16_hbm, ones_hbm,
       out_acc, out_deg, src_v, dst_v) = refs[:11]
      rows = refs[11:11 + _NRB]
      stage_v, acc_s = refs[11 + _NRB:13 + _NRB]
      sems = refs[13 + _NRB:13 + 2 * _NRB]
      ones_v, dstage_v, deg_s = refs[13 + 2 * _NRB:]
    else:
      (hlo_hbm, hhi_hbm, src_hbm, dst_hbm, zf_hbm,
       out_acc, src_v, dst_v) = refs[:8]
      rows = refs[8:8 + _NRB]
      stage_v, acc_s = refs[8 + _NRB:10 + _NRB]
      sems = refs[10 + _NRB:10 + 2 * _NRB]
    c = lax.axis_index("c")
    s = lax.axis_index("s")
    pltpu.sync_copy(zf_hbm, stage_v)
    if with_deg:
      pltpu.sync_copy(ones_hbm, ones_v)
      pltpu.sync_copy(z16_hbm, dstage_v)
    for r in range(_RPT // _SRG):
      pltpu.sync_copy(stage_v, acc_s.at[pl.ds(s * _RPT + r * _SRG, _SRG)])
      if with_deg:
        pltpu.sync_copy(dstage_v, deg_s.at[pl.ds(s * _RPT + r * _SRG, _SRG)])
    plsc.subcore_barrier()

    def pipeline(h_hbm):
      for b in range(_NB):
        blk = s * _NB + b
        pltpu.sync_copy(src_hbm.at[blk], src_v)
        pltpu.sync_copy(dst_hbm.at[blk], dst_v)
        for k in range(_NRB):
          pltpu.async_copy(h_hbm.at[src_v.at[k]], rows[k], sems[k])

        def grp(i, carry):
          for k in range(_NRB):
            g = _NRB * i + k
            pltpu.make_async_copy(h_hbm.at[src_v.at[0]], rows[k],
                                  sems[k]).wait()
            pltpu.sync_copy(rows[k], acc_s.at[dst_v.at[g]], add=True)
            if with_deg:
              pltpu.sync_copy(ones_v, deg_s.at[dst_v.at[g]], add=True)

            @pl.when(g + _NRB < _BS)
            def _():
              pltpu.async_copy(h_hbm.at[src_v.at[g + _NRB]], rows[k], sems[k])
          return carry

        lax.fori_loop(0, _BS // _NRB, grp, 0)

    @pl.when(c == 0)
    def _():
      pipeline(hlo_hbm)

    @pl.when(c == 1)
    def _():
      pipeline(hhi_hbm)
    plsc.subcore_barrier()
    orow = c * _NP + s * _RPT
    for r in range(_RPT // _SRG):
      pltpu.sync_copy(acc_s.at[pl.ds(s * _RPT + r * _SRG, _SRG)], stage_v)
      pltpu.sync_copy(stage_v, out_acc.at[pl.ds(orow + r * _SRG, _SRG)])
      if with_deg:
        pltpu.sync_copy(deg_s.at[pl.ds(s * _RPT + r * _SRG, _SRG)], dstage_v)
        pltpu.sync_copy(dstage_v, out_deg.at[pl.ds(orow + r * _SRG, _SRG)])

  mesh = plsc.VectorSubcoreMesh(core_axis_name="c", subcore_axis_name="s")
  return pl.kernel(body, out_type=tuple(out_type), mesh=mesh,
                   scratch_types=tuple(scratch),
                   compiler_params=pltpu.CompilerParams(
                       use_tc_tiling_on_sc=False))


def _dot_t(a, w):
  return lax.dot_general(a, w, (((1,), (1,)), ((), ())),
                         preferred_element_type=jnp.float32)


def _agg_mean(accp_ref, degp_ref):
  degp = degp_ref[...]
  deg = jnp.sum(degp[:_N], axis=1) * 0.0625
  inv = 1.0 / jnp.maximum(deg, 1.0)
  accp = accp_ref[...]
  agg = jnp.concatenate([accp[:_N], accp[_NP:_NP + _N]], axis=1)
  return agg * inv[:, None]


def _tc_layer1(accp_ref, degp_ref, x_ref, wl_ref, bl_ref, wr_ref, out_ref):
  agg = _agg_mean(accp_ref, degp_ref)
  h = _dot_t(agg, wl_ref[...]) + bl_ref[...] + _dot_t(x_ref[...], wr_ref[...])
  out_ref[...] = jnp.maximum(h, 0.0)


def _tc_final(accp_ref, degp_ref, h1_ref, wl_ref, bl_ref, wr_ref,
              tf_ref, wt1_ref, bt1_ref, wt2_ref, bt2_ref,
              wc1_ref, bc1_ref, wc2_ref, bc2_ref, out_ref):
  agg = _agg_mean(accp_ref, degp_ref)
  h2 = jnp.maximum(
      _dot_t(agg, wl_ref[...]) + bl_ref[...] + _dot_t(h1_ref[...], wr_ref[...]),
      0.0)
  t = _dot_t(jnp.maximum(_dot_t(tf_ref[...], wt1_ref[...]) + bt1_ref[...], 0.0),
             wt2_ref[...]) + bt2_ref[...]
  wc1 = wc1_ref[...]
  g = _dot_t(h2, wc1[:, :_D])
  cb = _dot_t(t, wc1[:, _D:]) + bc1_ref[...]
  wc2 = wc2_ref[...]
  cols = []
  for b in range(_B):
    hid = jnp.maximum(g + cb[b:b + 1, :], 0.0)
    cols.append(_dot_t(hid, wc2))
  out_ref[...] = jnp.concatenate(cols, axis=1) + bc2_ref[0, 0]


def kernel(x, edge_index, task_feat, W_l1, b_l1, W_r1, W_l2, b_l2, W_r2,
           Wt1, bt1, Wt2, bt2, Wc1, bc1, Wc2, bc2):
  src = edge_index[0].reshape(_NS * _NB, _BS, _CH)
  dst = edge_index[1].reshape(_NS * _NB, _BS, _CH)
  zf = jnp.zeros((_SRG, _HD), jnp.float32)
  z16 = jnp.zeros((_SRG, 16), jnp.float32)
  ones = jnp.ones((_CH, 16), jnp.float32)

  acc1, degp = _make_sc_agg(True)(
      x[:, :_HD], x[:, _HD:], src, dst, zf, z16, ones)

  h1 = pl.pallas_call(
      _tc_layer1,
      out_shape=jax.ShapeDtypeStruct((_N, _D), jnp.float32),
  )(acc1, degp, x, W_l1, b_l1.reshape(1, _D), W_r1)

  (acc2,) = _make_sc_agg(False)(h1[:, :_HD], h1[:, _HD:], src, dst, zf)

  scores_t = pl.pallas_call(
      _tc_final,
      out_shape=jax.ShapeDtypeStruct((_N, _B), jnp.float32),
  )(acc2, degp, h1, W_l2, b_l2.reshape(1, _D), W_r2,
    task_feat, Wt1, bt1.reshape(1, _D), Wt2, bt2.reshape(1, _D),
    Wc1, bc1.reshape(1, _D), Wc2, bc2.reshape(1, 1))

  return scores_t.T


# speedup vs baseline: 10.3679x; 1.0232x over previous
import functools

import jax
import jax.numpy as jnp
from jax import lax
from jax.experimental import pallas as pl
from jax.experimental.pallas import tpu as pltpu
from jax.experimental.pallas import tpu_sc as plsc

_N = 10000
_E = 320000
_D = 128
_B = 8

_NC = 2
_NS = 16
_HD = _D // _NC
_NP = 10240
_RPT = _NP // _NS
_EPT = _E // _NS
_CH = 80
_STEPS = _EPT // _CH
_NB = 5
_BS = _STEPS // _NB
_NRB = 5
_SRG = 128


def _make_sc_agg(with_deg):
  out_type = [jax.ShapeDtypeStruct((_NC * _NP, _HD), jnp.float32)]
  scratch = (
      [pltpu.VMEM((_BS, _CH), jnp.int32),
       pltpu.VMEM((_BS, _CH), jnp.int32)]
      + [pltpu.VMEM((_CH, _HD), jnp.float32)] * _NRB
      + [pltpu.VMEM((_SRG, _HD), jnp.float32),
         pltpu.VMEM_SHARED((_NP, _HD), jnp.float32)]
      + [pltpu.SemaphoreType.DMA] * _NRB
  )
  if with_deg:
    out_type.append(jax.ShapeDtypeStruct((_NC * _NP, 16), jnp.float32))
    scratch.append(pltpu.VMEM((_CH, 16), jnp.float32))
    scratch.append(pltpu.VMEM((_SRG, 16), jnp.float32))
    scratch.append(pltpu.VMEM_SHARED((_NP, 16), jnp.float32))

  def body(*refs):
    if with_deg:
      (hlo_hbm, hhi_hbm, src_hbm, dst_hbm, zf_hbm, z16_hbm, ones_hbm,
       out_acc, out_deg, src_v, dst_v) = refs[:11]
      rows = refs[11:11 + _NRB]
      stage_v, acc_s = refs[11 + _NRB:13 + _NRB]
      sems = refs[13 + _NRB:13 + 2 * _NRB]
      ones_v, dstage_v, deg_s = refs[13 + 2 * _NRB:]
    else:
      (hlo_hbm, hhi_hbm, src_hbm, dst_hbm, zf_hbm,
       out_acc, src_v, dst_v) = refs[:8]
      rows = refs[8:8 + _NRB]
      stage_v, acc_s = refs[8 + _NRB:10 + _NRB]
      sems = refs[10 + _NRB:10 + 2 * _NRB]
    c = lax.axis_index("c")
    s = lax.axis_index("s")
    pltpu.sync_copy(zf_hbm, stage_v)
    if with_deg:
      pltpu.sync_copy(ones_hbm, ones_v)
      pltpu.sync_copy(z16_hbm, dstage_v)
    for r in range(_RPT // _SRG):
      pltpu.sync_copy(stage_v, acc_s.at[pl.ds(s * _RPT + r * _SRG, _SRG)])
      if with_deg:
        pltpu.sync_copy(dstage_v, deg_s.at[pl.ds(s * _RPT + r * _SRG, _SRG)])
    plsc.subcore_barrier()

    def pipeline(h_hbm, deg_blocks=()):
      for b in range(_NB):
        blk = s * _NB + b
        pltpu.sync_copy(src_hbm.at[blk], src_v)
        pltpu.sync_copy(dst_hbm.at[blk], dst_v)
        for k in range(_NRB):
          pltpu.async_copy(h_hbm.at[src_v.at[k]], rows[k], sems[k])

        do_deg = with_deg and (b in deg_blocks)

        def grp(i, carry):
          for k in range(_NRB):
            g = _NRB * i + k
            pltpu.make_async_copy(h_hbm.at[src_v.at[0]], rows[k],
                                  sems[k]).wait()
            pltpu.sync_copy(rows[k], acc_s.at[dst_v.at[g]], add=True)
            if do_deg:
              pltpu.sync_copy(ones_v, deg_s.at[dst_v.at[g]], add=True)

            @pl.when(g + _NRB < _BS)
            def _():
              pltpu.async_copy(h_hbm.at[src_v.at[g + _NRB]], rows[k], sems[k])
          return carry

        lax.fori_loop(0, _BS // _NRB, grp, 0)

    @pl.when(c == 0)
    def _():
      pipeline(hlo_hbm, deg_blocks=(0, 1, 2))

    @pl.when(c == 1)
    def _():
      pipeline(hhi_hbm, deg_blocks=(3, 4))
    plsc.subcore_barrier()
    orow = c * _NP + s * _RPT
    for r in range(_RPT // _SRG):
      pltpu.sync_copy(acc_s.at[pl.ds(s * _RPT + r * _SRG, _SRG)], stage_v)
      pltpu.sync_copy(stage_v, out_acc.at[pl.ds(orow + r * _SRG, _SRG)])
      if with_deg:
        pltpu.sync_copy(deg_s.at[pl.ds(s * _RPT + r * _SRG, _SRG)], dstage_v)
        pltpu.sync_copy(dstage_v, out_deg.at[pl.ds(orow + r * _SRG, _SRG)])

  mesh = plsc.VectorSubcoreMesh(core_axis_name="c", subcore_axis_name="s")
  return pl.kernel(body, out_type=tuple(out_type), mesh=mesh,
                   scratch_types=tuple(scratch),
                   compiler_params=pltpu.CompilerParams(
                       use_tc_tiling_on_sc=False))


def _dot_t(a, w):
  return lax.dot_general(a, w, (((1,), (1,)), ((), ())),
                         preferred_element_type=jnp.float32)


def _agg_mean(accp_ref, degp_ref):
  degp = degp_ref[...]
  deg = jnp.sum(degp[:_N] + degp[_NP:_NP + _N], axis=1) * 0.0625
  inv = 1.0 / jnp.maximum(deg, 1.0)
  accp = accp_ref[...]
  agg = jnp.concatenate([accp[:_N], accp[_NP:_NP + _N]], axis=1)
  return agg * inv[:, None]


def _tc_layer1(accp_ref, degp_ref, x_ref, wl_ref, bl_ref, wr_ref, out_ref):
  agg = _agg_mean(accp_ref, degp_ref)
  h = _dot_t(agg, wl_ref[...]) + bl_ref[...] + _dot_t(x_ref[...], wr_ref[...])
  out_ref[...] = jnp.maximum(h, 0.0)


def _tc_final(accp_ref, degp_ref, h1_ref, wl_ref, bl_ref, wr_ref,
              tf_ref, wt1_ref, bt1_ref, wt2_ref, bt2_ref,
              wc1_ref, bc1_ref, wc2_ref, bc2_ref, out_ref):
  agg = _agg_mean(accp_ref, degp_ref)
  h2 = jnp.maximum(
      _dot_t(agg, wl_ref[...]) + bl_ref[...] + _dot_t(h1_ref[...], wr_ref[...]),
      0.0)
  t = _dot_t(jnp.maximum(_dot_t(tf_ref[...], wt1_ref[...]) + bt1_ref[...], 0.0),
             wt2_ref[...]) + bt2_ref[...]
  wc1 = wc1_ref[...]
  g = _dot_t(h2, wc1[:, :_D])
  cb = _dot_t(t, wc1[:, _D:]) + bc1_ref[...]
  wc2 = wc2_ref[...]
  cols = []
  for b in range(_B):
    hid = jnp.maximum(g + cb[b:b + 1, :], 0.0)
    cols.append(_dot_t(hid, wc2))
  out_ref[...] = jnp.concatenate(cols, axis=1) + bc2_ref[0, 0]


def kernel(x, edge_index, task_feat, W_l1, b_l1, W_r1, W_l2, b_l2, W_r2,
           Wt1, bt1, Wt2, bt2, Wc1, bc1, Wc2, bc2):
  src = edge_index[0].reshape(_NS * _NB, _BS, _CH)
  dst = edge_index[1].reshape(_NS * _NB, _BS, _CH)
  zf = jnp.zeros((_SRG, _HD), jnp.float32)
  z16 = jnp.zeros((_SRG, 16), jnp.float32)
  ones = jnp.ones((_CH, 16), jnp.float32)

  acc1, degp = _make_sc_agg(True)(
      x[:, :_HD], x[:, _HD:], src, dst, zf, z16, ones)

  h1 = pl.pallas_call(
      _tc_layer1,
      out_shape=jax.ShapeDtypeStruct((_N, _D), jnp.float32),
  )(acc1, degp, x, W_l1, b_l1.reshape(1, _D), W_r1)

  (acc2,) = _make_sc_agg(False)(h1[:, :_HD], h1[:, _HD:], src, dst, zf)

  scores_t = pl.pallas_call(
      _tc_final,
      out_shape=jax.ShapeDtypeStruct((_N, _B), jnp.float32),
  )(acc2, degp, h1, W_l2, b_l2.reshape(1, _D), W_r2,
    task_feat, Wt1, bt1.reshape(1, _D), Wt2, bt2.reshape(1, _D),
    Wc1, bc1.reshape(1, _D), Wc2, bc2.reshape(1, 1))

  return scores_t.T
